# Initial kernel scaffold; baseline (speedup 1.0000x reference)
import jax
import jax.numpy as jnp
from jax.experimental import pallas as pl


def kernel(x, segment_ids):
    raise NotImplementedError("write your pallas kernel here")


# speedup vs baseline: 2.3064x; 2.3064x over previous
import functools

import jax
import jax.numpy as jnp
from jax import lax
from jax.experimental import pallas as pl
from jax.experimental.pallas import tpu as pltpu
from jax.experimental.pallas import tpu_sc as plsc

N_NODES = 50000
D_FEAT = 512
NUM_SEGS = 256

NC = 2
NS = 16
L = 16
HALF = D_FEAT // NC
NCH = HALF // L
ROWS_PER_TILE = 3128
ROWS_LAST = N_NODES - (NS - 1) * ROWS_PER_TILE
BLK = 64
FULL_BLKS = ROWS_LAST // BLK
REM_MAIN = ROWS_PER_TILE - FULL_BLKS * BLK
REM_LAST = ROWS_LAST - FULL_BLKS * BLK

_mesh = plsc.VectorSubcoreMesh(core_axis_name="c", subcore_axis_name="s")


@functools.partial(
    pl.kernel,
    out_type=jax.ShapeDtypeStruct((NUM_SEGS, D_FEAT), jnp.float32),
    mesh=_mesh,
    scratch_types=[
        pltpu.VMEM((BLK, HALF), jnp.float32),
        pltpu.VMEM((BLK,), jnp.int32),
        pltpu.VMEM((NUM_SEGS, HALF), jnp.float32),
        pltpu.MemorySpace.HBM((NC, NS, NUM_SEGS, HALF), jnp.float32),
    ],
)
def _seg_sum_sc(x_hbm, ids_hbm, out_hbm, rowbuf, idxbuf, partial, shared):
    cid = lax.axis_index("c")
    sid = lax.axis_index("s")
    col0 = cid * HALF
    base = sid * ROWS_PER_TILE

    zeros16 = jnp.zeros((L,), jnp.float32)

    def _zrow(r, _):
        for j in range(NCH):
            partial[r, pl.ds(j * L, L)] = zeros16
        return 0

    lax.fori_loop(0, NUM_SEGS, _zrow, 0)

    def _accum_group(g0, nrows):
        v = idxbuf[pl.ds(g0, L)]
        for r in range(nrows):
            seg = v[r]
            for j in range(NCH):
                sl = pl.ds(j * L, L)
                plsc.addupdate(partial.at[seg, sl], rowbuf[g0 + r, sl])

    def _block(b, _):
        row0 = base + b * BLK
        pltpu.sync_copy(ids_hbm.at[pl.ds(row0, BLK)], idxbuf)
        pltpu.sync_copy(x_hbm.at[pl.ds(row0, BLK), pl.ds(col0, HALF)], rowbuf)

        def _group(g, _):
            _accum_group(g * L, L)
            return 0

        lax.fori_loop(0, BLK // L, _group, 0)
        return 0

    lax.fori_loop(0, FULL_BLKS, _block, 0)

    rem0 = base + FULL_BLKS * BLK

    @pl.when(sid < NS - 1)
    def _():
        pltpu.sync_copy(ids_hbm.at[pl.ds(rem0, REM_MAIN)],
                        idxbuf.at[pl.ds(0, REM_MAIN)])
        pltpu.sync_copy(x_hbm.at[pl.ds(rem0, REM_MAIN), pl.ds(col0, HALF)],
                        rowbuf.at[pl.ds(0, REM_MAIN)])
        for g in range(REM_MAIN // L):
            _accum_group(g * L, L)
        _accum_group((REM_MAIN // L) * L, REM_MAIN % L)

    @pl.when(sid == NS - 1)
    def _():
        pltpu.sync_copy(ids_hbm.at[pl.ds(rem0, REM_LAST)],
                        idxbuf.at[pl.ds(0, REM_LAST)])
        pltpu.sync_copy(x_hbm.at[pl.ds(rem0, REM_LAST), pl.ds(col0, HALF)],
                        rowbuf.at[pl.ds(0, REM_LAST)])
        _accum_group(0, REM_LAST)

    pltpu.sync_copy(partial, shared.at[cid, sid])
    plsc.subcore_barrier()

    r0 = sid * L
    acc = partial.at[pl.ds(0, L)]
    pltpu.sync_copy(shared.at[cid, 0, pl.ds(r0, L)], acc)

    def _comb(t, _):
        pltpu.sync_copy(shared.at[cid, t, pl.ds(r0, L)],
                        rowbuf.at[pl.ds(0, L)])
        for i in range(L):
            for j in range(NCH):
                sl = pl.ds(j * L, L)
                plsc.addupdate(partial.at[i, sl], rowbuf[i, sl])
        return 0

    lax.fori_loop(1, NS, _comb, 0)
    pltpu.sync_copy(acc, out_hbm.at[pl.ds(r0, L), pl.ds(col0, HALF)])


def kernel(x, segment_ids):
    ids = segment_ids.astype(jnp.int32)
    return _seg_sum_sc(x, ids)


# speedup vs baseline: 3.0861x; 1.3381x over previous
import functools

import jax
import jax.numpy as jnp
from jax import lax
from jax.experimental import pallas as pl
from jax.experimental.pallas import tpu as pltpu
from jax.experimental.pallas import tpu_sc as plsc

N_NODES = 50000
D_FEAT = 512
NUM_SEGS = 256

NC = 2
NS = 16
L = 16
HALF = D_FEAT // NC
NCH = HALF // L
ROWS_PER_TILE = 3128
ROWS_LAST = N_NODES - (NS - 1) * ROWS_PER_TILE
BLK = 64
FULL_BLKS = ROWS_LAST // BLK
REM_MAIN = ROWS_PER_TILE - FULL_BLKS * BLK
REM_LAST = ROWS_LAST - FULL_BLKS * BLK

_mesh = plsc.VectorSubcoreMesh(core_axis_name="c", subcore_axis_name="s")


@functools.partial(
    pl.kernel,
    out_type=jax.ShapeDtypeStruct((NUM_SEGS, D_FEAT), jnp.float32),
    mesh=_mesh,
    scratch_types=[
        pltpu.VMEM((BLK, HALF), jnp.float32),
        pltpu.VMEM((BLK, HALF), jnp.float32),
        pltpu.VMEM((BLK,), jnp.int32),
        pltpu.VMEM((BLK,), jnp.int32),
        pltpu.VMEM((NUM_SEGS, HALF), jnp.float32),
        pltpu.MemorySpace.HBM((NC, NS, NUM_SEGS, HALF), jnp.float32),
        pltpu.SemaphoreType.DMA,
        pltpu.SemaphoreType.DMA,
    ],
)
def _seg_sum_sc(x_hbm, ids_hbm, out_hbm, rowbuf0, rowbuf1, idxbuf0, idxbuf1,
                partial, shared, sem0, sem1):
    rowbufs, idxbufs, sems = (rowbuf0, rowbuf1), (idxbuf0, idxbuf1), (sem0, sem1)
    rowbuf, idxbuf = rowbuf0, idxbuf0
    cid = lax.axis_index("c")
    sid = lax.axis_index("s")
    col0 = cid * HALF
    base = sid * ROWS_PER_TILE

    zeros16 = jnp.zeros((L,), jnp.float32)

    def _zrow(r, _):
        for j in range(NCH):
            partial[r, pl.ds(j * L, L)] = zeros16
        return 0

    lax.fori_loop(0, NUM_SEGS, _zrow, 0)

    def _accum_group(rb, ib, g0, nrows):
        v = ib[pl.ds(g0, L)]
        for r in range(nrows):
            seg = v[r]
            for j in range(NCH):
                sl = pl.ds(j * L, L)
                plsc.addupdate(partial.at[seg, sl], rb[g0 + r, sl])

    def _start_fetch(b, d):
        row0 = base + b * BLK
        pltpu.async_copy(ids_hbm.at[pl.ds(row0, BLK)], idxbufs[d], sems[d])
        pltpu.async_copy(x_hbm.at[pl.ds(row0, BLK), pl.ds(col0, HALF)],
                         rowbufs[d], sems[d])

    def _wait_fetch(b, d):
        row0 = base + b * BLK
        pltpu.make_async_copy(ids_hbm.at[pl.ds(row0, BLK)], idxbufs[d],
                              sems[d]).wait()
        pltpu.make_async_copy(x_hbm.at[pl.ds(row0, BLK), pl.ds(col0, HALF)],
                              rowbufs[d], sems[d]).wait()

    _start_fetch(0, 0)

    def _block_pair(p, _):
        for d in range(2):
            b = p * 2 + d
            _wait_fetch(b, d)

            @pl.when(b + 1 < FULL_BLKS)
            def _():
                _start_fetch(b + 1, 1 - d)

            def _group(g, _):
                _accum_group(rowbufs[d], idxbufs[d], g * L, L)
                return 0

            lax.fori_loop(0, BLK // L, _group, 0)
        return 0

    lax.fori_loop(0, FULL_BLKS // 2, _block_pair, 0)

    rem0 = base + FULL_BLKS * BLK

    @pl.when(sid < NS - 1)
    def _():
        pltpu.sync_copy(ids_hbm.at[pl.ds(rem0, REM_MAIN)],
                        idxbuf.at[pl.ds(0, REM_MAIN)])
        pltpu.sync_copy(x_hbm.at[pl.ds(rem0, REM_MAIN), pl.ds(col0, HALF)],
                        rowbuf.at[pl.ds(0, REM_MAIN)])
        for g in range(REM_MAIN // L):
            _accum_group(rowbuf, idxbuf, g * L, L)
        _accum_group(rowbuf, idxbuf, (REM_MAIN // L) * L, REM_MAIN % L)

    @pl.when(sid == NS - 1)
    def _():
        pltpu.sync_copy(ids_hbm.at[pl.ds(rem0, REM_LAST)],
                        idxbuf.at[pl.ds(0, REM_LAST)])
        pltpu.sync_copy(x_hbm.at[pl.ds(rem0, REM_LAST), pl.ds(col0, HALF)],
                        rowbuf.at[pl.ds(0, REM_LAST)])
        _accum_group(rowbuf, idxbuf, 0, REM_LAST)

    pltpu.sync_copy(partial, shared.at[cid, sid])
    plsc.subcore_barrier()

    r0 = sid * L
    acc = partial.at[pl.ds(0, L)]
    pltpu.sync_copy(shared.at[cid, 0, pl.ds(r0, L)], acc)

    def _comb(t, _):
        pltpu.sync_copy(shared.at[cid, t, pl.ds(r0, L)],
                        rowbuf.at[pl.ds(0, L)])
        for i in range(L):
            for j in range(NCH):
                sl = pl.ds(j * L, L)
                plsc.addupdate(partial.at[i, sl], rowbuf[i, sl])
        return 0

    lax.fori_loop(1, NS, _comb, 0)
    pltpu.sync_copy(acc, out_hbm.at[pl.ds(r0, L), pl.ds(col0, HALF)])


def kernel(x, segment_ids):
    ids = segment_ids.astype(jnp.int32)
    return _seg_sum_sc(x, ids)


# speedup vs baseline: 3.5419x; 1.1477x over previous
import functools

import jax
import jax.numpy as jnp
from jax import lax
from jax.experimental import pallas as pl
from jax.experimental.pallas import tpu as pltpu
from jax.experimental.pallas import tpu_sc as plsc

N_NODES = 50000
D_FEAT = 512
NUM_SEGS = 256

NC = 2
NS = 16
L = 16
HALF = D_FEAT // NC
NCH = HALF // L
ROWS_PER_TILE = 3128
ROWS_LAST = N_NODES - (NS - 1) * ROWS_PER_TILE
BLK = 64
FULL_BLKS = ROWS_LAST // BLK
REM_MAIN = ROWS_PER_TILE - FULL_BLKS * BLK
REM_LAST = ROWS_LAST - FULL_BLKS * BLK

_mesh = plsc.VectorSubcoreMesh(core_axis_name="c", subcore_axis_name="s")


@functools.partial(
    pl.kernel,
    out_type=jax.ShapeDtypeStruct((NUM_SEGS, D_FEAT), jnp.float32),
    mesh=_mesh,
    scratch_types=[
        pltpu.VMEM((BLK, HALF), jnp.float32),
        pltpu.VMEM((BLK, HALF), jnp.float32),
        pltpu.VMEM((BLK,), jnp.int32),
        pltpu.VMEM((BLK,), jnp.int32),
        pltpu.VMEM((NUM_SEGS, HALF), jnp.float32),
        pltpu.VMEM((HALF,), jnp.float32),
        pltpu.SMEM((1,), jnp.int32),
        pltpu.MemorySpace.HBM((NC, NS, NUM_SEGS, HALF), jnp.float32),
        pltpu.SemaphoreType.DMA,
        pltpu.SemaphoreType.DMA,
    ],
)
def _seg_sum_sc(x_hbm, ids_hbm, out_hbm, rowbuf0, rowbuf1, idxbuf0, idxbuf1,
                partial, accbuf, prev_ref, shared, sem0, sem1):
    rowbufs, idxbufs, sems = (rowbuf0, rowbuf1), (idxbuf0, idxbuf1), (sem0, sem1)
    rowbuf, idxbuf = rowbuf0, idxbuf0
    cid = lax.axis_index("c")
    sid = lax.axis_index("s")
    col0 = cid * HALF
    base = sid * ROWS_PER_TILE

    zeros16 = jnp.zeros((L,), jnp.float32)

    def _zrow(r, _):
        for j in range(NCH):
            partial[r, pl.ds(j * L, L)] = zeros16
        return 0

    lax.fori_loop(0, NUM_SEGS, _zrow, 0)

    def _accum_group(rb, ib, g0, nrows):
        v = ib[pl.ds(g0, L)]
        for r in range(nrows):
            seg = v[r]
            for j in range(NCH):
                sl = pl.ds(j * L, L)
                plsc.addupdate(partial.at[seg, sl], rb[g0 + r, sl])

    def _tree_sum(chunks):
        while len(chunks) > 1:
            nxt = [chunks[k] + chunks[k + 1] for k in range(0, len(chunks) - 1, 2)]
            if len(chunks) % 2:
                nxt.append(chunks[-1])
            chunks = nxt
        return chunks[0]

    def _flush_reset(pv):
        for j in range(NCH):
            sl = pl.ds(j * L, L)
            plsc.addupdate(partial.at[pv, sl], accbuf[sl])
            accbuf[sl] = zeros16

    def _group_update(rb, ib, g0):
        v = ib[pl.ds(g0, L)]
        first = v[0]
        last = v[L - 1]

        @pl.when(first == last)
        def _fast():
            pv = prev_ref[0]

            @pl.when(pv != first)
            def _():
                _flush_reset(pv)

            for j in range(NCH):
                sl = pl.ds(j * L, L)
                g = _tree_sum([rb[g0 + r, sl] for r in range(L)])
                plsc.addupdate(accbuf.at[sl], g)
            prev_ref[0] = first

        @pl.when(first != last)
        def _slow():
            for r in range(L):
                seg = v[r]
                pv = prev_ref[0]

                @pl.when(seg != pv)
                def _(pv=pv):
                    _flush_reset(pv)

                for j in range(NCH):
                    sl = pl.ds(j * L, L)
                    plsc.addupdate(accbuf.at[sl], rb[g0 + r, sl])
                prev_ref[0] = seg

    def _start_fetch(b, d):
        row0 = base + b * BLK
        pltpu.async_copy(ids_hbm.at[pl.ds(row0, BLK)], idxbufs[d], sems[d])
        pltpu.async_copy(x_hbm.at[pl.ds(row0, BLK), pl.ds(col0, HALF)],
                         rowbufs[d], sems[d])

    def _wait_fetch(b, d):
        row0 = base + b * BLK
        pltpu.make_async_copy(ids_hbm.at[pl.ds(row0, BLK)], idxbufs[d],
                              sems[d]).wait()
        pltpu.make_async_copy(x_hbm.at[pl.ds(row0, BLK), pl.ds(col0, HALF)],
                              rowbufs[d], sems[d]).wait()

    _start_fetch(0, 0)
    prev_ref[0] = 0
    for j in range(NCH):
        accbuf[pl.ds(j * L, L)] = zeros16

    def _block_pair(p, _):
        for d in range(2):
            b = p * 2 + d
            _wait_fetch(b, d)

            @pl.when(b + 1 < FULL_BLKS)
            def _(b=b, d=d):
                _start_fetch(b + 1, 1 - d)

            def _group(g, _, d=d):
                _group_update(rowbufs[d], idxbufs[d], g * L)
                return 0

            lax.fori_loop(0, BLK // L, _group, 0)
        return 0

    lax.fori_loop(0, FULL_BLKS // 2, _block_pair, 0)
    _flush_reset(prev_ref[0])

    rem0 = base + FULL_BLKS * BLK

    @pl.when(sid < NS - 1)
    def _():
        pltpu.sync_copy(ids_hbm.at[pl.ds(rem0, REM_MAIN)],
                        idxbuf.at[pl.ds(0, REM_MAIN)])
        pltpu.sync_copy(x_hbm.at[pl.ds(rem0, REM_MAIN), pl.ds(col0, HALF)],
                        rowbuf.at[pl.ds(0, REM_MAIN)])
        for g in range(REM_MAIN // L):
            _accum_group(rowbuf, idxbuf, g * L, L)
        _accum_group(rowbuf, idxbuf, (REM_MAIN // L) * L, REM_MAIN % L)

    @pl.when(sid == NS - 1)
    def _():
        pltpu.sync_copy(ids_hbm.at[pl.ds(rem0, REM_LAST)],
                        idxbuf.at[pl.ds(0, REM_LAST)])
        pltpu.sync_copy(x_hbm.at[pl.ds(rem0, REM_LAST), pl.ds(col0, HALF)],
                        rowbuf.at[pl.ds(0, REM_LAST)])
        _accum_group(rowbuf, idxbuf, 0, REM_LAST)

    pltpu.sync_copy(partial, shared.at[cid, sid])
    plsc.subcore_barrier()

    r0 = sid * L
    for t in range(NS):
        pltpu.async_copy(shared.at[cid, t, pl.ds(r0, L)],
                         partial.at[pl.ds(t * L, L)], sem0)
    for t in range(NS):
        pltpu.make_async_copy(shared.at[cid, t, pl.ds(r0, L)],
                              partial.at[pl.ds(t * L, L)], sem0).wait()

    def _comb(i, _):
        for j in range(NCH):
            sl = pl.ds(j * L, L)
            rowbuf[i, sl] = _tree_sum(
                [partial[t * L + i, sl] for t in range(NS)])
        return 0

    lax.fori_loop(0, L, _comb, 0)
    pltpu.sync_copy(rowbuf.at[pl.ds(0, L)],
                    out_hbm.at[pl.ds(r0, L), pl.ds(col0, HALF)])


def kernel(x, segment_ids):
    ids = segment_ids.astype(jnp.int32)
    return _seg_sum_sc(x, ids)


# speedup vs baseline: 4.8757x; 1.3765x over previous
import functools

import jax
import jax.numpy as jnp
from jax import lax
from jax.experimental import pallas as pl
from jax.experimental.pallas import tpu as pltpu
from jax.experimental.pallas import tpu_sc as plsc

N_NODES = 50000
D_FEAT = 512
NUM_SEGS = 256

NC = 2
NS = 16
L = 16
HALF = D_FEAT // NC
NCH = HALF // L
ROWS_PER_TILE = 3128
ROWS_LAST = N_NODES - (NS - 1) * ROWS_PER_TILE
BLK = 64
FULL_BLKS = ROWS_LAST // BLK
REM_MAIN = ROWS_PER_TILE - FULL_BLKS * BLK
REM_LAST = ROWS_LAST - FULL_BLKS * BLK

_mesh = plsc.VectorSubcoreMesh(core_axis_name="c", subcore_axis_name="s")


@functools.partial(
    pl.kernel,
    out_type=jax.ShapeDtypeStruct((NUM_SEGS, D_FEAT), jnp.float32),
    mesh=_mesh,
    scratch_types=[
        pltpu.VMEM((BLK, HALF), jnp.float32),
        pltpu.VMEM((BLK, HALF), jnp.float32),
        pltpu.VMEM((BLK,), jnp.int32),
        pltpu.VMEM((BLK,), jnp.int32),
        pltpu.VMEM((NUM_SEGS, HALF), jnp.float32),
        pltpu.MemorySpace.HBM((NC, NS, NUM_SEGS, HALF), jnp.float32),
        pltpu.SemaphoreType.DMA,
        pltpu.SemaphoreType.DMA,
    ],
)
def _seg_sum_sc(x_hbm, ids_hbm, out_hbm, rowbuf0, rowbuf1, idxbuf0, idxbuf1,
                partial, shared, sem0, sem1):
    rowbufs, idxbufs, sems = (rowbuf0, rowbuf1), (idxbuf0, idxbuf1), (sem0, sem1)
    rowbuf, idxbuf = rowbuf0, idxbuf0
    cid = lax.axis_index("c")
    sid = lax.axis_index("s")
    col0 = cid * HALF
    base = sid * ROWS_PER_TILE

    zeros16 = jnp.zeros((L,), jnp.float32)

    def _zrow(r, _):
        for j in range(NCH):
            partial[r, pl.ds(j * L, L)] = zeros16
        return 0

    lax.fori_loop(0, NUM_SEGS, _zrow, 0)

    def _accum_group(rb, ib, g0, nrows):
        v = ib[pl.ds(g0, L)]
        for r in range(nrows):
            seg = v[r]
            for j in range(NCH):
                sl = pl.ds(j * L, L)
                plsc.addupdate(partial.at[seg, sl], rb[g0 + r, sl])

    def _tree_sum(chunks):
        while len(chunks) > 1:
            nxt = [chunks[k] + chunks[k + 1] for k in range(0, len(chunks) - 1, 2)]
            if len(chunks) % 2:
                nxt.append(chunks[-1])
            chunks = nxt
        return chunks[0]

    def _group_update(rb, ib, g0):
        v = ib[pl.ds(g0, L)]
        first = v[0]
        last = v[L - 1]

        @pl.when(first == last)
        def _fast():
            gs = []
            loads = [rb[g0 + r, pl.ds(0, L)] for r in range(L)]
            for j in range(NCH):
                nxt = ([rb[g0 + r, pl.ds((j + 1) * L, L)] for r in range(L)]
                       if j + 1 < NCH else None)
                gs.append(_tree_sum(loads))
                loads = nxt
            for j in range(NCH):
                plsc.addupdate(partial.at[first, pl.ds(j * L, L)], gs[j])

        @pl.when(first != last)
        def _slow():
            _accum_group(rb, ib, g0, L)

    def _start_fetch(b, d):
        row0 = base + b * BLK
        pltpu.async_copy(ids_hbm.at[pl.ds(row0, BLK)], idxbufs[d], sems[d])
        pltpu.async_copy(x_hbm.at[pl.ds(row0, BLK), pl.ds(col0, HALF)],
                         rowbufs[d], sems[d])

    def _wait_fetch(b, d):
        row0 = base + b * BLK
        pltpu.make_async_copy(ids_hbm.at[pl.ds(row0, BLK)], idxbufs[d],
                              sems[d]).wait()
        pltpu.make_async_copy(x_hbm.at[pl.ds(row0, BLK), pl.ds(col0, HALF)],
                              rowbufs[d], sems[d]).wait()

    _start_fetch(0, 0)

    def _block_pair(p, _):
        for d in range(2):
            b = p * 2 + d
            _wait_fetch(b, d)

            @pl.when(b + 1 < FULL_BLKS)
            def _(b=b, d=d):
                _start_fetch(b + 1, 1 - d)

            def _group(g, _, d=d):
                _group_update(rowbufs[d], idxbufs[d], g * L)
                return 0

            lax.fori_loop(0, BLK // L, _group, 0)
        return 0

    lax.fori_loop(0, FULL_BLKS // 2, _block_pair, 0)

    rem0 = base + FULL_BLKS * BLK

    @pl.when(sid < NS - 1)
    def _():
        pltpu.sync_copy(ids_hbm.at[pl.ds(rem0, REM_MAIN)],
                        idxbuf.at[pl.ds(0, REM_MAIN)])
        pltpu.sync_copy(x_hbm.at[pl.ds(rem0, REM_MAIN), pl.ds(col0, HALF)],
                        rowbuf.at[pl.ds(0, REM_MAIN)])
        for g in range(REM_MAIN // L):
            _accum_group(rowbuf, idxbuf, g * L, L)
        _accum_group(rowbuf, idxbuf, (REM_MAIN // L) * L, REM_MAIN % L)

    @pl.when(sid == NS - 1)
    def _():
        pltpu.sync_copy(ids_hbm.at[pl.ds(rem0, REM_LAST)],
                        idxbuf.at[pl.ds(0, REM_LAST)])
        pltpu.sync_copy(x_hbm.at[pl.ds(rem0, REM_LAST), pl.ds(col0, HALF)],
                        rowbuf.at[pl.ds(0, REM_LAST)])
        _accum_group(rowbuf, idxbuf, 0, REM_LAST)

    pltpu.sync_copy(partial, shared.at[cid, sid])
    plsc.subcore_barrier()

    r0 = sid * L
    for t in range(NS):
        pltpu.async_copy(shared.at[cid, t, pl.ds(r0, L)],
                         partial.at[pl.ds(t * L, L)], sem0)
    for t in range(NS):
        pltpu.make_async_copy(shared.at[cid, t, pl.ds(r0, L)],
                              partial.at[pl.ds(t * L, L)], sem0).wait()

    def _comb(i, _):
        for j in range(NCH):
            sl = pl.ds(j * L, L)
            rowbuf[i, sl] = _tree_sum(
                [partial[t * L + i, sl] for t in range(NS)])
        return 0

    lax.fori_loop(0, L, _comb, 0)
    pltpu.sync_copy(rowbuf.at[pl.ds(0, L)],
                    out_hbm.at[pl.ds(r0, L), pl.ds(col0, HALF)])


def kernel(x, segment_ids):
    ids = segment_ids.astype(jnp.int32)
    return _seg_sum_sc(x, ids)


# speedup vs baseline: 6.4341x; 1.3196x over previous
import functools

import jax
import jax.numpy as jnp
from jax import lax
from jax.experimental import pallas as pl
from jax.experimental.pallas import tpu as pltpu
from jax.experimental.pallas import tpu_sc as plsc

N_NODES = 50000
D_FEAT = 512
NUM_SEGS = 256

NC = 2
NS = 16
L = 16
HALF = D_FEAT // NC
NCH = HALF // L
ROWS_PER_TILE = 3128
ROWS_LAST = N_NODES - (NS - 1) * ROWS_PER_TILE
BLK = 64
FULL_BLKS = ROWS_LAST // BLK
REM_MAIN = ROWS_PER_TILE - FULL_BLKS * BLK
REM_LAST = ROWS_LAST - FULL_BLKS * BLK

_mesh = plsc.VectorSubcoreMesh(core_axis_name="c", subcore_axis_name="s")


@functools.partial(
    pl.kernel,
    out_type=jax.ShapeDtypeStruct((NUM_SEGS, D_FEAT), jnp.float32),
    mesh=_mesh,
    scratch_types=[
        pltpu.VMEM((BLK, HALF), jnp.float32),
        pltpu.VMEM((BLK, HALF), jnp.float32),
        pltpu.VMEM((BLK,), jnp.int32),
        pltpu.VMEM((BLK,), jnp.int32),
        pltpu.VMEM((NUM_SEGS, HALF), jnp.float32),
        pltpu.MemorySpace.HBM((NC, NS, NUM_SEGS, HALF), jnp.float32),
        pltpu.SemaphoreType.DMA,
        pltpu.SemaphoreType.DMA,
    ],
)
def _seg_sum_sc(x_hbm, ids_hbm, out_hbm, rowbuf0, rowbuf1, idxbuf0, idxbuf1,
                partial, shared, sem0, sem1):
    rowbufs, idxbufs, sems = (rowbuf0, rowbuf1), (idxbuf0, idxbuf1), (sem0, sem1)
    rowbuf, idxbuf = rowbuf0, idxbuf0
    cid = lax.axis_index("c")
    sid = lax.axis_index("s")
    col0 = cid * HALF
    base = sid * ROWS_PER_TILE

    zeros16 = jnp.zeros((L,), jnp.float32)

    def _zrow(r, _):
        for j in range(NCH):
            partial[r, pl.ds(j * L, L)] = zeros16
        return 0

    lax.fori_loop(0, NUM_SEGS, _zrow, 0)

    def _accum_group(rb, ib, g0, nrows):
        v = ib[pl.ds(g0, L)]
        for r in range(nrows):
            seg = v[r]
            for j in range(NCH):
                sl = pl.ds(j * L, L)
                plsc.addupdate(partial.at[seg, sl], rb[g0 + r, sl])

    def _tree_sum(chunks):
        while len(chunks) > 1:
            nxt = [chunks[k] + chunks[k + 1] for k in range(0, len(chunks) - 1, 2)]
            if len(chunks) % 2:
                nxt.append(chunks[-1])
            chunks = nxt
        return chunks[0]

    def _group_update(rb, ib, g0):
        v = ib[pl.ds(g0, L)]
        first = v[0]
        last = v[L - 1]

        @pl.when(first == last)
        def _fast():
            loads = [rb[g0 + r, pl.ds(0, L)] for r in range(L)]
            for j in range(NCH):
                nxt = ([rb[g0 + r, pl.ds((j + 1) * L, L)] for r in range(L)]
                       if j + 1 < NCH else None)
                plsc.addupdate(partial.at[first, pl.ds(j * L, L)],
                               _tree_sum(loads))
                loads = nxt

        @pl.when(first != last)
        def _slow():
            _accum_group(rb, ib, g0, L)

    def _start_fetch(b, d):
        row0 = base + b * BLK
        pltpu.async_copy(ids_hbm.at[pl.ds(row0, BLK)], idxbufs[d], sems[d])
        pltpu.async_copy(x_hbm.at[pl.ds(row0, BLK), pl.ds(col0, HALF)],
                         rowbufs[d], sems[d])

    def _wait_fetch(b, d):
        row0 = base + b * BLK
        pltpu.make_async_copy(ids_hbm.at[pl.ds(row0, BLK)], idxbufs[d],
                              sems[d]).wait()
        pltpu.make_async_copy(x_hbm.at[pl.ds(row0, BLK), pl.ds(col0, HALF)],
                              rowbufs[d], sems[d]).wait()

    _start_fetch(0, 0)

    def _block_pair(p, _):
        for d in range(2):
            b = p * 2 + d
            _wait_fetch(b, d)

            @pl.when(b + 1 < FULL_BLKS)
            def _(b=b, d=d):
                _start_fetch(b + 1, 1 - d)

            def _group(g, _, d=d):
                _group_update(rowbufs[d], idxbufs[d], g * L)
                return 0

            lax.fori_loop(0, BLK // L, _group, 0)
        return 0

    lax.fori_loop(0, FULL_BLKS // 2, _block_pair, 0)

    rem0 = base + FULL_BLKS * BLK

    @pl.when(sid < NS - 1)
    def _():
        pltpu.sync_copy(ids_hbm.at[pl.ds(rem0, REM_MAIN)],
                        idxbuf.at[pl.ds(0, REM_MAIN)])
        pltpu.sync_copy(x_hbm.at[pl.ds(rem0, REM_MAIN), pl.ds(col0, HALF)],
                        rowbuf.at[pl.ds(0, REM_MAIN)])
        for g in range(REM_MAIN // L):
            _accum_group(rowbuf, idxbuf, g * L, L)
        _accum_group(rowbuf, idxbuf, (REM_MAIN // L) * L, REM_MAIN % L)

    @pl.when(sid == NS - 1)
    def _():
        pltpu.sync_copy(ids_hbm.at[pl.ds(rem0, REM_LAST)],
                        idxbuf.at[pl.ds(0, REM_LAST)])
        pltpu.sync_copy(x_hbm.at[pl.ds(rem0, REM_LAST), pl.ds(col0, HALF)],
                        rowbuf.at[pl.ds(0, REM_LAST)])
        _accum_group(rowbuf, idxbuf, 0, REM_LAST)

    pltpu.sync_copy(partial, shared.at[cid, sid])
    plsc.subcore_barrier()

    r0 = sid * L
    for t in range(NS):
        pltpu.async_copy(shared.at[cid, t, pl.ds(r0, L)],
                         partial.at[pl.ds(t * L, L)], sem0)
    for t in range(NS):
        pltpu.make_async_copy(shared.at[cid, t, pl.ds(r0, L)],
                              partial.at[pl.ds(t * L, L)], sem0).wait()

    def _comb(i, _):
        for j in range(NCH):
            sl = pl.ds(j * L, L)
            rowbuf[i, sl] = _tree_sum(
                [partial[t * L + i, sl] for t in range(NS)])
        return 0

    lax.fori_loop(0, L, _comb, 0)
    pltpu.sync_copy(rowbuf.at[pl.ds(0, L)],
                    out_hbm.at[pl.ds(r0, L), pl.ds(col0, HALF)])


def kernel(x, segment_ids):
    ids = segment_ids.astype(jnp.int32)
    return _seg_sum_sc(x, ids)


# speedup vs baseline: 6.5225x; 1.0137x over previous
import functools

import jax
import jax.numpy as jnp
from jax import lax
from jax.experimental import pallas as pl
from jax.experimental.pallas import tpu as pltpu
from jax.experimental.pallas import tpu_sc as plsc

N_NODES = 50000
D_FEAT = 512
NUM_SEGS = 256

NC = 2
NS = 16
L = 16
HALF = D_FEAT // NC
NCH = HALF // L
ROWS_PER_TILE = 3128
ROWS_LAST = N_NODES - (NS - 1) * ROWS_PER_TILE
BLK = 64
FULL_BLKS = ROWS_LAST // BLK
REM_MAIN = ROWS_PER_TILE - FULL_BLKS * BLK
REM_LAST = ROWS_LAST - FULL_BLKS * BLK

_mesh = plsc.VectorSubcoreMesh(core_axis_name="c", subcore_axis_name="s")


@functools.partial(
    pl.kernel,
    out_type=jax.ShapeDtypeStruct((NUM_SEGS, D_FEAT), jnp.float32),
    mesh=_mesh,
    scratch_types=[
        pltpu.VMEM((BLK, HALF), jnp.float32),
        pltpu.VMEM((BLK, HALF), jnp.float32),
        pltpu.VMEM((BLK,), jnp.int32),
        pltpu.VMEM((BLK,), jnp.int32),
        pltpu.VMEM((NUM_SEGS, HALF), jnp.float32),
        pltpu.MemorySpace.HBM((NC, NS, NUM_SEGS, HALF), jnp.float32),
        pltpu.SemaphoreType.DMA,
        pltpu.SemaphoreType.DMA,
    ],
)
def _seg_sum_sc(x_hbm, ids_hbm, out_hbm, rowbuf0, rowbuf1, idxbuf0, idxbuf1,
                partial, shared, sem0, sem1):
    rowbufs, idxbufs, sems = (rowbuf0, rowbuf1), (idxbuf0, idxbuf1), (sem0, sem1)
    rowbuf, idxbuf = rowbuf0, idxbuf0
    cid = lax.axis_index("c")
    sid = lax.axis_index("s")
    col0 = cid * HALF
    base = sid * ROWS_PER_TILE

    zeros16 = jnp.zeros((L,), jnp.float32)

    def _zrow(r, _):
        for j in range(NCH):
            partial[r, pl.ds(j * L, L)] = zeros16
        return 0

    lax.fori_loop(0, NUM_SEGS, _zrow, 0)

    def _accum_group(rb, ib, g0, nrows):
        v = ib[pl.ds(g0, L)]
        for r in range(nrows):
            seg = v[r]
            for j in range(NCH):
                sl = pl.ds(j * L, L)
                plsc.addupdate(partial.at[seg, sl], rb[g0 + r, sl])

    def _tree_sum(chunks):
        while len(chunks) > 1:
            nxt = [chunks[k] + chunks[k + 1] for k in range(0, len(chunks) - 1, 2)]
            if len(chunks) % 2:
                nxt.append(chunks[-1])
            chunks = nxt
        return chunks[0]

    def _group_update(rb, ib, g0):
        v = ib[pl.ds(g0, L)]
        first = v[0]
        last = v[L - 1]

        @pl.when(first == last)
        def _fast():
            loads = [rb[g0 + r, pl.ds(0, L)] for r in range(L)]
            for j in range(NCH):
                nxt = ([rb[g0 + r, pl.ds((j + 1) * L, L)] for r in range(L)]
                       if j + 1 < NCH else None)
                plsc.addupdate(partial.at[first, pl.ds(j * L, L)],
                               _tree_sum(loads))
                loads = nxt

        @pl.when(first != last)
        def _slow():
            has_mid = False
            for r in range(1, L - 1):
                has_mid = jnp.logical_or(
                    has_mid, jnp.logical_and(v[r] != first, v[r] != last))

            @pl.when(jnp.logical_not(has_mid))
            def _two_runs():
                ms = [v[r] == first for r in range(L)]
                loads = [rb[g0 + r, pl.ds(0, L)] for r in range(L)]
                for j in range(NCH):
                    nxt = ([rb[g0 + r, pl.ds((j + 1) * L, L)]
                            for r in range(L)] if j + 1 < NCH else None)
                    sl = pl.ds(j * L, L)
                    tf = _tree_sum(
                        [jnp.where(ms[r], loads[r], zeros16) for r in range(L)])
                    ta = _tree_sum(loads)
                    plsc.addupdate(partial.at[first, sl], tf)
                    plsc.addupdate(partial.at[last, sl], ta - tf)
                    loads = nxt

            @pl.when(has_mid)
            def _general():
                _accum_group(rb, ib, g0, L)

    def _start_fetch(b, d):
        row0 = base + b * BLK
        pltpu.async_copy(ids_hbm.at[pl.ds(row0, BLK)], idxbufs[d], sems[d])
        pltpu.async_copy(x_hbm.at[pl.ds(row0, BLK), pl.ds(col0, HALF)],
                         rowbufs[d], sems[d])

    def _wait_fetch(b, d):
        row0 = base + b * BLK
        pltpu.make_async_copy(ids_hbm.at[pl.ds(row0, BLK)], idxbufs[d],
                              sems[d]).wait()
        pltpu.make_async_copy(x_hbm.at[pl.ds(row0, BLK), pl.ds(col0, HALF)],
                              rowbufs[d], sems[d]).wait()

    _start_fetch(0, 0)

    def _block_pair(p, _):
        for d in range(2):
            b = p * 2 + d
            _wait_fetch(b, d)

            @pl.when(b + 1 < FULL_BLKS)
            def _(b=b, d=d):
                _start_fetch(b + 1, 1 - d)

            def _group(g, _, d=d):
                _group_update(rowbufs[d], idxbufs[d], g * L)
                return 0

            lax.fori_loop(0, BLK // L, _group, 0)
        return 0

    lax.fori_loop(0, FULL_BLKS // 2, _block_pair, 0)

    rem0 = base + FULL_BLKS * BLK

    @pl.when(sid < NS - 1)
    def _():
        pltpu.sync_copy(ids_hbm.at[pl.ds(rem0, REM_MAIN)],
                        idxbuf.at[pl.ds(0, REM_MAIN)])
        pltpu.sync_copy(x_hbm.at[pl.ds(rem0, REM_MAIN), pl.ds(col0, HALF)],
                        rowbuf.at[pl.ds(0, REM_MAIN)])
        for g in range(REM_MAIN // L):
            _group_update(rowbuf, idxbuf, g * L)
        _accum_group(rowbuf, idxbuf, (REM_MAIN // L) * L, REM_MAIN % L)

    @pl.when(sid == NS - 1)
    def _():
        pltpu.sync_copy(ids_hbm.at[pl.ds(rem0, REM_LAST)],
                        idxbuf.at[pl.ds(0, REM_LAST)])
        pltpu.sync_copy(x_hbm.at[pl.ds(rem0, REM_LAST), pl.ds(col0, HALF)],
                        rowbuf.at[pl.ds(0, REM_LAST)])
        _accum_group(rowbuf, idxbuf, 0, REM_LAST)

    pltpu.sync_copy(partial, shared.at[cid, sid])
    plsc.subcore_barrier()

    r0 = sid * L
    for t in range(NS):
        pltpu.async_copy(shared.at[cid, t, pl.ds(r0, L)],
                         partial.at[pl.ds(t * L, L)], sem0)
    for t in range(NS):
        pltpu.make_async_copy(shared.at[cid, t, pl.ds(r0, L)],
                              partial.at[pl.ds(t * L, L)], sem0).wait()

    def _comb(i, _):
        for j in range(NCH):
            sl = pl.ds(j * L, L)
            rowbuf[i, sl] = _tree_sum(
                [partial[t * L + i, sl] for t in range(NS)])
        return 0

    lax.fori_loop(0, L, _comb, 0)
    pltpu.sync_copy(rowbuf.at[pl.ds(0, L)],
                    out_hbm.at[pl.ds(r0, L), pl.ds(col0, HALF)])


def kernel(x, segment_ids):
    ids = segment_ids.astype(jnp.int32)
    return _seg_sum_sc(x, ids)
